# baseline (speedup 1.0000x reference)
import functools

import jax
import jax.numpy as jnp
import numpy as np
from jax import lax
from jax.experimental import pallas as pl
from jax.experimental.pallas import tpu as pltpu
from jax.experimental.pallas import tpu_sc as plsc

N = 10000
N_PAD = 10240
E = 320000
LANES = 128
N_SUB = 16
N_WORKERS = 2 * N_SUB
CHUNKS = 80
E_PAD = CHUNKS * LANES * N_WORKERS
ROWS_PER_SUB = N_PAD // N_SUB
IN_CH, HID_CH, OUT_CH = 128, 128, 64
PK = OUT_CH // 2
RB = 512

_PERM = np.zeros(OUT_CH, np.int32)
for _g in range(2):
    for _j in range(16):
        _PERM[32 * _g + 2 * _j] = 32 * _g + _j
        _PERM[32 * _g + 2 * _j + 1] = 32 * _g + 16 + _j
_INV = np.argsort(_PERM)

_mesh = plsc.VectorSubcoreMesh(core_axis_name="c", subcore_axis_name="s")


@functools.partial(
    pl.kernel,
    out_type=jax.ShapeDtypeStruct((N_WORKERS, N_PAD), jnp.float32),
    mesh=_mesh,
    scratch_types=[
        pltpu.VMEM((CHUNKS, LANES), jnp.int32),
        pltpu.VMEM((N_PAD,), jnp.float32),
    ],
    compiler_params=pltpu.CompilerParams(needs_layout_passes=False),
)
def _sc_degree(dst_hbm, out_hbm, dst_v, hist_v):
    c = lax.axis_index("c")
    s = lax.axis_index("s")
    wid = c * N_SUB + s
    pltpu.sync_copy(dst_hbm.at[pl.ds(wid * CHUNKS, CHUNKS)], dst_v)
    zero16 = jnp.zeros((16,), jnp.float32)

    def zbody(i, carry):
        hist_v[pl.ds(i * 16, 16)] = zero16
        return carry

    lax.fori_loop(0, N_PAD // 16, zbody, 0)
    ones16 = jnp.ones((16,), jnp.float32)

    def body(j, carry):
        for k in range(LANES // 16):
            idx = dst_v[j, pl.ds(k * 16, 16)]
            plsc.addupdate_scatter(hist_v, [idx], ones16)
        return carry

    lax.fori_loop(0, CHUNKS, body, 0)
    pltpu.sync_copy(hist_v, out_hbm.at[wid])


NBUF = 2
PCH = 80
_MASK_HI = jnp.int32(-65536)


def _widen(rb, fb):
    def crow(r, carry):
        for g in range(2):
            v = rb[r, pl.ds(16 * g, 16)]
            fb[r, pl.ds(32 * g, 16)] = plsc.bitcast(lax.shift_left(v, 16), jnp.float32)
            fb[r, pl.ds(32 * g + 16, 16)] = plsc.bitcast(lax.bitwise_and(v, _MASK_HI), jnp.float32)
        return carry

    lax.fori_loop(0, LANES, crow, 0)


def _seed_from_packed(hp_hbm, acc, rb, fb, rs):
    def piece(p, carry):
        pltpu.sync_copy(hp_hbm.at[pl.ds(rs + p * LANES, LANES)], rb)
        _widen(rb, fb)
        pltpu.sync_copy(fb, acc.at[pl.ds(rs + p * LANES, LANES)])
        return carry

    lax.fori_loop(0, ROWS_PER_SUB // LANES, piece, 0)


def _edge_loop(hp_s, acc, src_hbm, dst_hbm, src_v, dst_v, rows, fbufs, sems, ssems, base, phases):
    for phase in range(phases):
        pltpu.sync_copy(src_hbm.at[pl.ds(base + phase * PCH, PCH)], src_v)
        pltpu.sync_copy(dst_hbm.at[pl.ds(base + phase * PCH, PCH)], dst_v)
        for b in range(NBUF):
            pltpu.async_copy(hp_s.at[src_v.at[b]], rows[b], sems[b])

        def body(t, carry):
            j = t * NBUF
            for b in range(NBUF):
                pltpu.make_async_copy(hp_s.at[src_v.at[j + b]], rows[b], sems[b]).wait()

                @pl.when(j + b >= NBUF)
                def _():
                    pltpu.make_async_copy(fbufs[b], acc.at[dst_v.at[j + b - NBUF]], ssems[b]).wait()

                _widen(rows[b], fbufs[b])
                pltpu.async_copy(fbufs[b], acc.at[dst_v.at[j + b]], ssems[b], add=True)

                @pl.when(j + b + NBUF < PCH)
                def _():
                    pltpu.async_copy(hp_s.at[src_v.at[j + b + NBUF]], rows[b], sems[b])

            return carry

        lax.fori_loop(0, PCH // NBUF, body, 0)
        for b in range(NBUF):
            pltpu.make_async_copy(fbufs[b], acc.at[dst_v.at[PCH - NBUF + b]], ssems[b]).wait()


_SC_PROP_SCRATCH = [
    pltpu.VMEM((PCH, LANES), jnp.int32),
    pltpu.VMEM((PCH, LANES), jnp.int32),
    [pltpu.VMEM((LANES, PK), jnp.int32) for _ in range(NBUF)],
    [pltpu.VMEM((LANES, OUT_CH), jnp.float32) for _ in range(NBUF)],
    pltpu.VMEM_SHARED((N_PAD, OUT_CH), jnp.float32),
    pltpu.VMEM_SHARED((N_PAD, PK), jnp.int32),
    [pltpu.SemaphoreType.DMA for _ in range(NBUF)],
    [pltpu.SemaphoreType.DMA for _ in range(NBUF)],
]


@functools.partial(
    pl.kernel,
    out_type=jax.ShapeDtypeStruct((2, N_PAD, OUT_CH), jnp.float32),
    mesh=_mesh,
    scratch_types=_SC_PROP_SCRATCH,
    compiler_params=pltpu.CompilerParams(use_tc_tiling_on_sc=False, needs_layout_passes=False),
)
def _sc_prop_l1(pa_hbm, pb_hbm, src_hbm, dst_hbm, out_hbm, src_v, dst_v, rows, fbufs, acc, hp_s, sems, ssems):
    c = lax.axis_index("c")
    s = lax.axis_index("s")
    rs = s * ROWS_PER_SUB

    @pl.when(c == 0)
    def _():
        pltpu.sync_copy(pa_hbm.at[pl.ds(rs, ROWS_PER_SUB)], hp_s.at[pl.ds(rs, ROWS_PER_SUB)])
        _seed_from_packed(pa_hbm, acc, rows[0], fbufs[0], rs)

    @pl.when(c != 0)
    def _():
        pltpu.sync_copy(pb_hbm.at[pl.ds(rs, ROWS_PER_SUB)], hp_s.at[pl.ds(rs, ROWS_PER_SUB)])
        _seed_from_packed(pb_hbm, acc, rows[0], fbufs[0], rs)

    plsc.subcore_barrier()
    _edge_loop(hp_s, acc, src_hbm, dst_hbm, src_v, dst_v, rows, fbufs, sems, ssems, s * (2 * CHUNKS), 2)
    plsc.subcore_barrier()
    pltpu.sync_copy(acc.at[pl.ds(rs, ROWS_PER_SUB)], out_hbm.at[c, pl.ds(rs, ROWS_PER_SUB)])


@functools.partial(
    pl.kernel,
    out_type=jax.ShapeDtypeStruct((2, N_PAD, OUT_CH), jnp.float32),
    mesh=_mesh,
    scratch_types=_SC_PROP_SCRATCH,
    compiler_params=pltpu.CompilerParams(use_tc_tiling_on_sc=False, needs_layout_passes=False),
)
def _sc_prop(hp_hbm, zero_hbm, src_hbm, dst_hbm, out_hbm, src_v, dst_v, rows, fbufs, acc, hp_s, sems, ssems):
    c = lax.axis_index("c")
    s = lax.axis_index("s")
    wid = c * N_SUB + s
    rs = s * ROWS_PER_SUB

    pltpu.sync_copy(hp_hbm.at[pl.ds(rs, ROWS_PER_SUB)], hp_s.at[pl.ds(rs, ROWS_PER_SUB)])

    @pl.when(c == 0)
    def _():
        _seed_from_packed(hp_hbm, acc, rows[0], fbufs[0], rs)

    @pl.when(c != 0)
    def _():
        pltpu.sync_copy(zero_hbm.at[pl.ds(rs, ROWS_PER_SUB)], acc.at[pl.ds(rs, ROWS_PER_SUB)])

    plsc.subcore_barrier()
    _edge_loop(hp_s, acc, src_hbm, dst_hbm, src_v, dst_v, rows, fbufs, sems, ssems, wid * CHUNKS, 1)
    plsc.subcore_barrier()
    pltpu.sync_copy(acc.at[pl.ds(rs, ROWS_PER_SUB)], out_hbm.at[c, pl.ds(rs, ROWS_PER_SUB)])


def _dis_from_parts(degp):
    deg = jnp.sum(degp, axis=0) + 1.0
    return lax.rsqrt(deg)[:, None]


def _pack_bf16(h):
    hb = h.astype(jnp.bfloat16)
    return lax.bitcast_convert_type(hb.reshape(h.shape[0], PK, 2), jnp.int32)


def _tc_k1(x_ref, w_ref, degp_ref, outa_ref, outb_ref):
    dis = _dis_from_parts(degp_ref[...])
    h = jnp.dot(x_ref[...], w_ref[...], preferred_element_type=jnp.float32) * dis
    outa_ref[...] = h[:, :OUT_CH].astype(jnp.bfloat16)
    outb_ref[...] = h[:, OUT_CH:].astype(jnp.bfloat16)


def _tc_k2(p_ref, degp_ref, b1_ref, w2_ref, out_ref):
    dis = _dis_from_parts(degp_ref[...])
    ssum = jnp.concatenate([p_ref[0], p_ref[1]], axis=1)
    h = jnp.maximum(ssum * dis + b1_ref[...], 0.0)
    out_ref[...] = (jnp.dot(h, w2_ref[...], preferred_element_type=jnp.float32) * dis).astype(jnp.bfloat16)


def _tc_k3(q_ref, degp_ref, b2_ref, out_ref):
    dis = _dis_from_parts(degp_ref[...])
    out_ref[...] = (q_ref[0] + q_ref[1]) * dis + b2_ref[...]


def kernel(x, edge_index, W1, b1, W2, b2):
    x_pad = jnp.zeros((N_PAD, IN_CH), jnp.float32).at[:N].set(x)
    src = edge_index[0].astype(jnp.int32)
    dst = edge_index[1].astype(jnp.int32)
    pad = jnp.full((E_PAD - E,), N, jnp.int32)
    src_r = jnp.concatenate([src, pad]).reshape(E_PAD // LANES, LANES)
    dst_r = jnp.concatenate([dst, pad]).reshape(E_PAD // LANES, LANES)
    zeros_o = jnp.zeros((N_PAD, OUT_CH), jnp.float32)

    inv128 = np.concatenate([_INV, OUT_CH + _INV])
    b1p = b1[inv128]
    w2p = W2[inv128, :]
    b2p = b2[_INV]

    degp = _sc_degree(dst_r)

    grid = (N_PAD // RB,)
    h1a, h1b = pl.pallas_call(
        _tc_k1,
        grid=grid,
        in_specs=[
            pl.BlockSpec((RB, IN_CH), lambda i: (i, 0)),
            pl.BlockSpec((IN_CH, HID_CH), lambda i: (0, 0)),
            pl.BlockSpec((N_WORKERS, RB), lambda i: (0, i)),
        ],
        out_specs=[
            pl.BlockSpec((RB, OUT_CH), lambda i: (i, 0)),
            pl.BlockSpec((RB, OUT_CH), lambda i: (i, 0)),
        ],
        out_shape=[
            jax.ShapeDtypeStruct((N_PAD, OUT_CH), jnp.bfloat16),
            jax.ShapeDtypeStruct((N_PAD, OUT_CH), jnp.bfloat16),
        ],
    )(x_pad, W1, degp)

    part1 = _sc_prop_l1(_pack_bf16(h1a), _pack_bf16(h1b), src_r, dst_r)

    h2f = pl.pallas_call(
        _tc_k2,
        grid=grid,
        in_specs=[
            pl.BlockSpec((2, RB, OUT_CH), lambda i: (0, i, 0)),
            pl.BlockSpec((N_WORKERS, RB), lambda i: (0, i)),
            pl.BlockSpec((1, HID_CH), lambda i: (0, 0)),
            pl.BlockSpec((HID_CH, OUT_CH), lambda i: (0, 0)),
        ],
        out_specs=pl.BlockSpec((RB, OUT_CH), lambda i: (i, 0)),
        out_shape=jax.ShapeDtypeStruct((N_PAD, OUT_CH), jnp.bfloat16),
    )(part1, degp, b1p[None, :], w2p)

    part2 = _sc_prop(_pack_bf16(h2f), zeros_o, src_r, dst_r)

    zp = pl.pallas_call(
        _tc_k3,
        grid=grid,
        in_specs=[
            pl.BlockSpec((2, RB, OUT_CH), lambda i: (0, i, 0)),
            pl.BlockSpec((N_WORKERS, RB), lambda i: (0, i)),
            pl.BlockSpec((1, OUT_CH), lambda i: (0, 0)),
        ],
        out_specs=pl.BlockSpec((RB, OUT_CH), lambda i: (i, 0)),
        out_shape=jax.ShapeDtypeStruct((N_PAD, OUT_CH), jnp.float32),
    )(part2, degp, b2p[None, :])

    return zp[:N, _PERM]


# speedup vs baseline: 2.2193x; 2.2193x over previous
import functools

import jax
import jax.numpy as jnp
from jax import lax
from jax.experimental import pallas as pl
from jax.experimental.pallas import tpu as pltpu
from jax.experimental.pallas import tpu_sc as plsc

N = 10000
N_PAD = 10240
E = 320000
LANES = 128
N_SUB = 16
N_WORKERS = 2 * N_SUB
CHUNKS = 80
E_PAD = CHUNKS * LANES * N_WORKERS
ROWS_PER_SUB = N_PAD // N_SUB
IN_CH, HID_CH, OUT_CH = 128, 128, 64
RB = 512

_mesh = plsc.VectorSubcoreMesh(core_axis_name="c", subcore_axis_name="s")


@functools.partial(
    pl.kernel,
    out_type=jax.ShapeDtypeStruct((N_WORKERS, N_PAD), jnp.float32),
    mesh=_mesh,
    scratch_types=[
        pltpu.VMEM((CHUNKS, LANES), jnp.int32),
        pltpu.VMEM((N_PAD,), jnp.float32),
    ],
    compiler_params=pltpu.CompilerParams(needs_layout_passes=False),
)
def _sc_degree(dst_hbm, out_hbm, dst_v, hist_v):
    c = lax.axis_index("c")
    s = lax.axis_index("s")
    wid = c * N_SUB + s
    pltpu.sync_copy(dst_hbm.at[pl.ds(wid * CHUNKS, CHUNKS)], dst_v)
    zero16 = jnp.zeros((16,), jnp.float32)

    def zbody(i, carry):
        hist_v[pl.ds(i * 16, 16)] = zero16
        return carry

    lax.fori_loop(0, N_PAD // 16, zbody, 0)
    ones16 = jnp.ones((16,), jnp.float32)

    def body(j, carry):
        for k in range(LANES // 16):
            idx = dst_v[j, pl.ds(k * 16, 16)]
            plsc.addupdate_scatter(hist_v, [idx], ones16)
        return carry

    lax.fori_loop(0, CHUNKS, body, 0)
    pltpu.sync_copy(hist_v, out_hbm.at[wid])


NBUF = 2
PCH = 80


def _edge_loop(hp_s, acc, src_hbm, dst_hbm, src_v, dst_v, rows, sems, base, phases):
    for phase in range(phases):
        pltpu.sync_copy(src_hbm.at[pl.ds(base + phase * PCH, PCH)], src_v)
        pltpu.sync_copy(dst_hbm.at[pl.ds(base + phase * PCH, PCH)], dst_v)
        for b in range(NBUF):
            pltpu.async_copy(hp_s.at[src_v.at[b]], rows[b], sems[b])

        def body(t, carry):
            j = t * NBUF
            for b in range(NBUF):
                pltpu.make_async_copy(hp_s.at[src_v.at[j + b]], rows[b], sems[b]).wait()
                pltpu.sync_copy(rows[b], acc.at[dst_v.at[j + b]], add=True)

                @pl.when(j + b + NBUF < PCH)
                def _():
                    pltpu.async_copy(hp_s.at[src_v.at[j + b + NBUF]], rows[b], sems[b])

            return carry

        lax.fori_loop(0, PCH // NBUF, body, 0)


_SC_PROP_SCRATCH = [
    pltpu.VMEM((PCH, LANES), jnp.int32),
    pltpu.VMEM((PCH, LANES), jnp.int32),
    [pltpu.VMEM((LANES, OUT_CH), jnp.float32) for _ in range(NBUF)],
    pltpu.VMEM_SHARED((N_PAD, OUT_CH), jnp.float32),
    pltpu.VMEM_SHARED((N_PAD, OUT_CH), jnp.float32),
    [pltpu.SemaphoreType.DMA for _ in range(NBUF)],
]


@functools.partial(
    pl.kernel,
    out_type=jax.ShapeDtypeStruct((2, N_PAD, OUT_CH), jnp.float32),
    mesh=_mesh,
    scratch_types=_SC_PROP_SCRATCH,
    compiler_params=pltpu.CompilerParams(use_tc_tiling_on_sc=False),
)
def _sc_prop_l1(ha_hbm, hb_hbm, src_hbm, dst_hbm, out_hbm, src_v, dst_v, rows, acc, hp_s, sems):
    c = lax.axis_index("c")
    s = lax.axis_index("s")
    rs = s * ROWS_PER_SUB

    @pl.when(c == 0)
    def _():
        pltpu.sync_copy(ha_hbm.at[pl.ds(rs, ROWS_PER_SUB)], hp_s.at[pl.ds(rs, ROWS_PER_SUB)])
        pltpu.sync_copy(ha_hbm.at[pl.ds(rs, ROWS_PER_SUB)], acc.at[pl.ds(rs, ROWS_PER_SUB)])

    @pl.when(c != 0)
    def _():
        pltpu.sync_copy(hb_hbm.at[pl.ds(rs, ROWS_PER_SUB)], hp_s.at[pl.ds(rs, ROWS_PER_SUB)])
        pltpu.sync_copy(hb_hbm.at[pl.ds(rs, ROWS_PER_SUB)], acc.at[pl.ds(rs, ROWS_PER_SUB)])

    plsc.subcore_barrier()
    _edge_loop(hp_s, acc, src_hbm, dst_hbm, src_v, dst_v, rows, sems, s * (2 * CHUNKS), 2)
    plsc.subcore_barrier()
    pltpu.sync_copy(acc.at[pl.ds(rs, ROWS_PER_SUB)], out_hbm.at[c, pl.ds(rs, ROWS_PER_SUB)])


@functools.partial(
    pl.kernel,
    out_type=jax.ShapeDtypeStruct((2, N_PAD, OUT_CH), jnp.float32),
    mesh=_mesh,
    scratch_types=_SC_PROP_SCRATCH,
    compiler_params=pltpu.CompilerParams(use_tc_tiling_on_sc=False),
)
def _sc_prop(hp_hbm, zero_hbm, src_hbm, dst_hbm, out_hbm, src_v, dst_v, rows, acc, hp_s, sems):
    c = lax.axis_index("c")
    s = lax.axis_index("s")
    wid = c * N_SUB + s
    rs = s * ROWS_PER_SUB

    pltpu.sync_copy(hp_hbm.at[pl.ds(rs, ROWS_PER_SUB)], hp_s.at[pl.ds(rs, ROWS_PER_SUB)])

    @pl.when(c == 0)
    def _():
        pltpu.sync_copy(hp_hbm.at[pl.ds(rs, ROWS_PER_SUB)], acc.at[pl.ds(rs, ROWS_PER_SUB)])

    @pl.when(c != 0)
    def _():
        pltpu.sync_copy(zero_hbm.at[pl.ds(rs, ROWS_PER_SUB)], acc.at[pl.ds(rs, ROWS_PER_SUB)])

    plsc.subcore_barrier()
    _edge_loop(hp_s, acc, src_hbm, dst_hbm, src_v, dst_v, rows, sems, wid * CHUNKS, 1)
    plsc.subcore_barrier()
    pltpu.sync_copy(acc.at[pl.ds(rs, ROWS_PER_SUB)], out_hbm.at[c, pl.ds(rs, ROWS_PER_SUB)])


def _dis_from_parts(degp):
    deg = jnp.sum(degp, axis=0) + 1.0
    return lax.rsqrt(deg)[:, None]


def _tc_k1(x_ref, w_ref, degp_ref, outa_ref, outb_ref):
    dis = _dis_from_parts(degp_ref[...])
    h = jnp.dot(x_ref[...], w_ref[...], preferred_element_type=jnp.float32) * dis
    outa_ref[...] = h[:, :OUT_CH]
    outb_ref[...] = h[:, OUT_CH:]


def _tc_k2(p_ref, degp_ref, b1_ref, w2_ref, out_ref):
    dis = _dis_from_parts(degp_ref[...])
    ssum = jnp.concatenate([p_ref[0], p_ref[1]], axis=1)
    h = jnp.maximum(ssum * dis + b1_ref[...], 0.0)
    out_ref[...] = jnp.dot(h, w2_ref[...], preferred_element_type=jnp.float32) * dis


def _tc_k3(q_ref, degp_ref, b2_ref, out_ref):
    dis = _dis_from_parts(degp_ref[...])
    out_ref[...] = (q_ref[0] + q_ref[1]) * dis + b2_ref[...]


def kernel(x, edge_index, W1, b1, W2, b2):
    x_pad = jnp.zeros((N_PAD, IN_CH), jnp.float32).at[:N].set(x)
    src = edge_index[0].astype(jnp.int32)
    dst = edge_index[1].astype(jnp.int32)
    pad = jnp.full((E_PAD - E,), N, jnp.int32)
    src_r = jnp.concatenate([src, pad]).reshape(E_PAD // LANES, LANES)
    dst_r = jnp.concatenate([dst, pad]).reshape(E_PAD // LANES, LANES)
    zeros_o = jnp.zeros((N_PAD, OUT_CH), jnp.float32)

    degp = _sc_degree(dst_r)

    grid = (N_PAD // RB,)
    h1a, h1b = pl.pallas_call(
        _tc_k1,
        grid=grid,
        in_specs=[
            pl.BlockSpec((RB, IN_CH), lambda i: (i, 0)),
            pl.BlockSpec((IN_CH, HID_CH), lambda i: (0, 0)),
            pl.BlockSpec((N_WORKERS, RB), lambda i: (0, i)),
        ],
        out_specs=[
            pl.BlockSpec((RB, OUT_CH), lambda i: (i, 0)),
            pl.BlockSpec((RB, OUT_CH), lambda i: (i, 0)),
        ],
        out_shape=[
            jax.ShapeDtypeStruct((N_PAD, OUT_CH), jnp.float32),
            jax.ShapeDtypeStruct((N_PAD, OUT_CH), jnp.float32),
        ],
    )(x_pad, W1, degp)

    part1 = _sc_prop_l1(h1a, h1b, src_r, dst_r)

    h2p = pl.pallas_call(
        _tc_k2,
        grid=grid,
        in_specs=[
            pl.BlockSpec((2, RB, OUT_CH), lambda i: (0, i, 0)),
            pl.BlockSpec((N_WORKERS, RB), lambda i: (0, i)),
            pl.BlockSpec((1, HID_CH), lambda i: (0, 0)),
            pl.BlockSpec((HID_CH, OUT_CH), lambda i: (0, 0)),
        ],
        out_specs=pl.BlockSpec((RB, OUT_CH), lambda i: (i, 0)),
        out_shape=jax.ShapeDtypeStruct((N_PAD, OUT_CH), jnp.float32),
    )(part1, degp, b1[None, :], W2)

    part2 = _sc_prop(h2p, zeros_o, src_r, dst_r)

    z = pl.pallas_call(
        _tc_k3,
        grid=grid,
        in_specs=[
            pl.BlockSpec((2, RB, OUT_CH), lambda i: (0, i, 0)),
            pl.BlockSpec((N_WORKERS, RB), lambda i: (0, i)),
            pl.BlockSpec((1, OUT_CH), lambda i: (0, 0)),
        ],
        out_specs=pl.BlockSpec((RB, OUT_CH), lambda i: (i, 0)),
        out_shape=jax.ShapeDtypeStruct((N_PAD, OUT_CH), jnp.float32),
    )(part2, degp, b2[None, :])

    return z[:N]
